# baseline (speedup 1.0000x reference)
import functools

import jax
import jax.numpy as jnp
from jax import lax
from jax.experimental import pallas as pl
from jax.experimental.pallas import tpu as pltpu
from jax.experimental.pallas import tpu_sc as plsc

D = 64
DP = 128
NSEQ = 4096
T = 200
B = NSEQ * T
NC, NS = 2, 16
NW = NC * NS
SPW = NSEQ // NW
BPW = SPW * T
NBUF = 4


@functools.cache
def _build():
  mesh = plsc.VectorSubcoreMesh(core_axis_name="c", subcore_axis_name="s")

  @functools.partial(
      pl.kernel,
      mesh=mesh,
      out_type=jax.ShapeDtypeStruct((NSEQ, T, DP), jnp.float32),
      compiler_params=pltpu.CompilerParams(
          use_tc_tiling_on_sc=True, skip_device_barrier=True),
      scratch_types=[
          pltpu.VMEM((BPW,), jnp.int32),
          [pltpu.VMEM((T, DP), jnp.float32) for _ in range(NBUF)],
          [pltpu.SemaphoreType.DMA for _ in range(NBUF)],
          [pltpu.SemaphoreType.DMA for _ in range(NBUF)],
      ],
  )
  def emb(w_hbm, ids_hbm, out_hbm, idx_v, rows, gsem, osem):
    wid = lax.axis_index("s") * NC + lax.axis_index("c")
    seq0 = wid * SPW
    pltpu.sync_copy(ids_hbm.at[pl.ds(wid * BPW, BPW)], idx_v)

    def gathers(g, bi):
      off = g * T
      return [
          pltpu.make_async_copy(
              w_hbm.at[idx_v.at[pl.ds(off, 128)]],
              rows[bi].at[pl.ds(0, 128)],
              gsem[bi],
          ),
          pltpu.make_async_copy(
              w_hbm.at[idx_v.at[pl.ds(off + 128, 72)]],
              rows[bi].at[pl.ds(128, 72)],
              gsem[bi],
          ),
      ]

    def out_copy(g, bi):
      return pltpu.make_async_copy(
          rows[bi], out_hbm.at[seq0 + g], osem[bi])

    def do_group(g, bi, first=False, start_next=True):
      for d in gathers(g, bi):
        d.wait()
      out_copy(g, bi).start()
      nbi = (bi + 3) % NBUF
      if not first:
        out_copy(g - 1, nbi).wait()
      if start_next:
        for d in gathers(g + 3, nbi):
          d.start()

    for g in range(3):
      for d in gathers(g, g):
        d.start()

    do_group(0, 0, first=True)
    for k in range(1, NBUF):
      do_group(k, k)

    def body(gg, carry):
      g0 = gg * NBUF
      for k in range(NBUF):
        do_group(g0 + k, k)
      return carry

    lax.fori_loop(1, SPW // NBUF - 1, body, 0)

    g0 = SPW - NBUF
    for k in range(NBUF):
      do_group(g0 + k, k, start_next=(g0 + k + 3 < SPW))
    out_copy(SPW - 1, (SPW - 1) % NBUF).wait()

  return emb


def kernel(ids, W):
  ids_flat = ids.reshape(-1).astype(jnp.int32)
  w_padded = jnp.concatenate([W, W], axis=1)
  return _build()(w_padded, ids_flat)[:, :, :D]


# speedup vs baseline: 1.2227x; 1.2227x over previous
import functools

import jax
import jax.numpy as jnp
from jax import lax
from jax.experimental import pallas as pl
from jax.experimental.pallas import tpu as pltpu
from jax.experimental.pallas import tpu_sc as plsc

V = 1_000_000
D = 64
DP = 128
NSEQ = 4096
T = 200
B = NSEQ * T
NC, NS = 2, 16
NW = NC * NS
SPW = NSEQ // NW
BPW = SPW * T
NBUF = 4
TBLK = 2048


@functools.cache
def _build_widen():
  def body(wt_ref, out_ref):
    t = jnp.transpose(wt_ref[...], (1, 0))
    out_ref[...] = jnp.pad(t, ((0, 0), (0, DP - D)))

  return pl.pallas_call(
      body,
      grid=(pl.cdiv(V, TBLK),),
      in_specs=[pl.BlockSpec((D, TBLK), lambda i: (0, i))],
      out_specs=pl.BlockSpec((TBLK, DP), lambda i: (i, 0)),
      out_shape=jax.ShapeDtypeStruct((V, DP), jnp.float32),
  )


@functools.cache
def _build_gather():
  mesh = plsc.VectorSubcoreMesh(core_axis_name="c", subcore_axis_name="s")

  @functools.partial(
      pl.kernel,
      mesh=mesh,
      out_type=jax.ShapeDtypeStruct((NSEQ, T, DP), jnp.float32),
      compiler_params=pltpu.CompilerParams(
          use_tc_tiling_on_sc=True, skip_device_barrier=True),
      scratch_types=[
          pltpu.VMEM((BPW,), jnp.int32),
          [pltpu.VMEM((T, DP), jnp.float32) for _ in range(NBUF)],
          [pltpu.SemaphoreType.DMA for _ in range(NBUF)],
          [pltpu.SemaphoreType.DMA for _ in range(NBUF)],
      ],
  )
  def emb(w_hbm, ids_hbm, out_hbm, idx_v, rows, gsem, osem):
    wid = lax.axis_index("s") * NC + lax.axis_index("c")
    seq0 = wid * SPW
    pltpu.sync_copy(ids_hbm.at[pl.ds(wid * BPW, BPW)], idx_v)

    def gathers(g, bi):
      off = g * T
      return [
          pltpu.make_async_copy(
              w_hbm.at[idx_v.at[pl.ds(off, 128)]],
              rows[bi].at[pl.ds(0, 128)],
              gsem[bi],
          ),
          pltpu.make_async_copy(
              w_hbm.at[idx_v.at[pl.ds(off + 128, 72)]],
              rows[bi].at[pl.ds(128, 72)],
              gsem[bi],
          ),
      ]

    def out_copy(g, bi):
      return pltpu.make_async_copy(
          rows[bi], out_hbm.at[seq0 + g], osem[bi])

    def do_group(g, bi, first=False, start_next=True):
      for d in gathers(g, bi):
        d.wait()
      out_copy(g, bi).start()
      nbi = (bi + 3) % NBUF
      if not first:
        out_copy(g - 1, nbi).wait()
      if start_next:
        for d in gathers(g + 3, nbi):
          d.start()

    for g in range(3):
      for d in gathers(g, g):
        d.start()

    do_group(0, 0, first=True)
    for k in range(1, NBUF):
      do_group(k, k)

    def body(gg, carry):
      g0 = gg * NBUF
      for k in range(NBUF):
        do_group(g0 + k, k)
      return carry

    lax.fori_loop(1, SPW // NBUF - 1, body, 0)

    g0 = SPW - NBUF
    for k in range(NBUF):
      do_group(g0 + k, k, start_next=(g0 + k + 3 < SPW))
    out_copy(SPW - 1, (SPW - 1) % NBUF).wait()

  return emb


def kernel(ids, W):
  ids_flat = ids.reshape(-1).astype(jnp.int32)
  w_padded = _build_widen()(W.T)
  return _build_gather()(w_padded, ids_flat)[:, :, :D]


# speedup vs baseline: 1.4098x; 1.1530x over previous
import functools

import jax
import jax.numpy as jnp
from jax import lax
from jax.experimental import pallas as pl
from jax.experimental.pallas import tpu as pltpu
from jax.experimental.pallas import tpu_sc as plsc

V = 1_000_000
D = 64
DP = 128
NSEQ = 4096
T = 200
B = NSEQ * T
NC, NS = 2, 16
NW = NC * NS
SPW = NSEQ // NW
BPW = SPW * T
NBUF = 4
TBLK = 4096


@functools.cache
def _build_widen():
  def body(wt_ref, out_ref):
    out_ref[:, : D] = jnp.transpose(wt_ref[...], (1, 0))

  return pl.pallas_call(
      body,
      grid=(pl.cdiv(V, TBLK),),
      in_specs=[pl.BlockSpec((D, TBLK), lambda i: (0, i))],
      out_specs=pl.BlockSpec((TBLK, DP), lambda i: (i, 0)),
      out_shape=jax.ShapeDtypeStruct((V, DP), jnp.float32),
  )


@functools.cache
def _build_gather():
  mesh = plsc.VectorSubcoreMesh(core_axis_name="c", subcore_axis_name="s")

  @functools.partial(
      pl.kernel,
      mesh=mesh,
      out_type=jax.ShapeDtypeStruct((NSEQ, T, DP), jnp.float32),
      compiler_params=pltpu.CompilerParams(
          use_tc_tiling_on_sc=True, skip_device_barrier=True),
      scratch_types=[
          pltpu.VMEM((BPW,), jnp.int32),
          [pltpu.VMEM((T, DP), jnp.float32) for _ in range(NBUF)],
          [pltpu.SemaphoreType.DMA for _ in range(NBUF)],
          [pltpu.SemaphoreType.DMA for _ in range(NBUF)],
      ],
  )
  def emb(w_hbm, ids_hbm, out_hbm, idx_v, rows, gsem, osem):
    wid = lax.axis_index("s") * NC + lax.axis_index("c")
    seq0 = wid * SPW
    pltpu.sync_copy(ids_hbm.at[pl.ds(wid * BPW, BPW)], idx_v)

    def gathers(g, bi):
      off = g * T
      return [
          pltpu.make_async_copy(
              w_hbm.at[idx_v.at[pl.ds(off, 128)]],
              rows[bi].at[pl.ds(0, 128)],
              gsem[bi],
          ),
          pltpu.make_async_copy(
              w_hbm.at[idx_v.at[pl.ds(off + 128, 72)]],
              rows[bi].at[pl.ds(128, 72)],
              gsem[bi],
          ),
      ]

    def out_copy(g, bi):
      return pltpu.make_async_copy(
          rows[bi], out_hbm.at[seq0 + g], osem[bi])

    def do_group(g, bi, first=False, start_next=True):
      for d in gathers(g, bi):
        d.wait()
      out_copy(g, bi).start()
      nbi = (bi + 3) % NBUF
      if not first:
        out_copy(g - 1, nbi).wait()
      if start_next:
        for d in gathers(g + 3, nbi):
          d.start()

    for g in range(3):
      for d in gathers(g, g):
        d.start()

    do_group(0, 0, first=True)
    for k in range(1, NBUF):
      do_group(k, k)

    def body(gg, carry):
      g0 = gg * NBUF
      for k in range(NBUF):
        do_group(g0 + k, k)
      return carry

    lax.fori_loop(1, SPW // NBUF - 1, body, 0)

    g0 = SPW - NBUF
    for k in range(NBUF):
      do_group(g0 + k, k, start_next=(g0 + k + 3 < SPW))
    out_copy(SPW - 1, (SPW - 1) % NBUF).wait()

  return emb


def kernel(ids, W):
  ids_flat = ids.reshape(-1).astype(jnp.int32)
  w_padded = _build_widen()(W.T)
  return _build_gather()(w_padded, ids_flat)[:, :, :D]


# speedup vs baseline: 1.5434x; 1.0948x over previous
import functools

import jax
import jax.numpy as jnp
from jax import lax
from jax.experimental import pallas as pl
from jax.experimental.pallas import tpu as pltpu
from jax.experimental.pallas import tpu_sc as plsc

V = 1_000_000
D = 64
DP = 128
NSEQ = 4096
T = 200
B = NSEQ * T
NC, NS = 2, 16
NW = NC * NS
SPW = NSEQ // NW
BPW = SPW * T
NBUF = 4
TBLK = 8192


@functools.cache
def _build_widen():
  def body(wt_ref, out_ref):
    out_ref[:, : D] = jnp.transpose(wt_ref[...], (1, 0))

  return pl.pallas_call(
      body,
      grid=(pl.cdiv(V, TBLK),),
      in_specs=[pl.BlockSpec((D, TBLK), lambda i: (0, i))],
      out_specs=pl.BlockSpec((TBLK, DP), lambda i: (i, 0)),
      out_shape=jax.ShapeDtypeStruct((V, DP), jnp.float32),
  )


@functools.cache
def _build_gather():
  mesh = plsc.VectorSubcoreMesh(core_axis_name="c", subcore_axis_name="s")

  @functools.partial(
      pl.kernel,
      mesh=mesh,
      out_type=jax.ShapeDtypeStruct((NSEQ, T, DP), jnp.float32),
      compiler_params=pltpu.CompilerParams(
          use_tc_tiling_on_sc=True, skip_device_barrier=True),
      scratch_types=[
          pltpu.VMEM((BPW,), jnp.int32),
          [pltpu.VMEM((T, DP), jnp.float32) for _ in range(NBUF)],
          [pltpu.SemaphoreType.DMA for _ in range(NBUF)],
          [pltpu.SemaphoreType.DMA for _ in range(NBUF)],
      ],
  )
  def emb(w_hbm, ids_hbm, out_hbm, idx_v, rows, gsem, osem):
    wid = lax.axis_index("s") * NC + lax.axis_index("c")
    seq0 = wid * SPW
    pltpu.sync_copy(ids_hbm.at[pl.ds(wid * BPW, BPW)], idx_v)

    def gathers(g, bi):
      off = g * T
      return [
          pltpu.make_async_copy(
              w_hbm.at[idx_v.at[pl.ds(off, 128)]],
              rows[bi].at[pl.ds(0, 128)],
              gsem[bi],
          ),
          pltpu.make_async_copy(
              w_hbm.at[idx_v.at[pl.ds(off + 128, 72)]],
              rows[bi].at[pl.ds(128, 72)],
              gsem[bi],
          ),
      ]

    def out_copy(g, bi):
      return pltpu.make_async_copy(
          rows[bi], out_hbm.at[seq0 + g], osem[bi])

    def do_group(g, bi, first=False, start_next=True):
      for d in gathers(g, bi):
        d.wait()
      out_copy(g, bi).start()
      nbi = (bi + 3) % NBUF
      if not first:
        out_copy(g - 1, nbi).wait()
      if start_next:
        for d in gathers(g + 3, nbi):
          d.start()

    for g in range(3):
      for d in gathers(g, g):
        d.start()

    do_group(0, 0, first=True)
    for k in range(1, NBUF):
      do_group(k, k)

    def body(gg, carry):
      g0 = gg * NBUF
      for k in range(NBUF):
        do_group(g0 + k, k)
      return carry

    lax.fori_loop(1, SPW // NBUF - 1, body, 0)

    g0 = SPW - NBUF
    for k in range(NBUF):
      do_group(g0 + k, k, start_next=(g0 + k + 3 < SPW))
    out_copy(SPW - 1, (SPW - 1) % NBUF).wait()

  return emb


def kernel(ids, W):
  ids_flat = ids.reshape(-1).astype(jnp.int32)
  w_padded = _build_widen()(W.T)
  return _build_gather()(w_padded, ids_flat)[:, :, :D]


# speedup vs baseline: 1.5837x; 1.0262x over previous
import functools

import jax
import jax.numpy as jnp
from jax import lax
from jax.experimental import pallas as pl
from jax.experimental.pallas import tpu as pltpu
from jax.experimental.pallas import tpu_sc as plsc

V = 1_000_000
D = 64
DP = 128
NSEQ = 4096
T = 200
B = NSEQ * T
NC, NS = 2, 16
NW = NC * NS
SPW = NSEQ // NW
BPW = SPW * T
NBUF = 4
TBLK = 16384


@functools.cache
def _build_widen():
  def body(wt_ref, out_ref):
    out_ref[:, : D] = jnp.transpose(wt_ref[...], (1, 0))

  return pl.pallas_call(
      body,
      grid=(pl.cdiv(V, TBLK),),
      in_specs=[pl.BlockSpec((D, TBLK), lambda i: (0, i))],
      out_specs=pl.BlockSpec((TBLK, DP), lambda i: (i, 0)),
      out_shape=jax.ShapeDtypeStruct((V, DP), jnp.float32),
  )


@functools.cache
def _build_gather():
  mesh = plsc.VectorSubcoreMesh(core_axis_name="c", subcore_axis_name="s")

  @functools.partial(
      pl.kernel,
      mesh=mesh,
      out_type=jax.ShapeDtypeStruct((NSEQ, T, DP), jnp.float32),
      compiler_params=pltpu.CompilerParams(
          use_tc_tiling_on_sc=True, skip_device_barrier=True),
      scratch_types=[
          pltpu.VMEM((BPW,), jnp.int32),
          [pltpu.VMEM((T, DP), jnp.float32) for _ in range(NBUF)],
          [pltpu.SemaphoreType.DMA for _ in range(NBUF)],
          [pltpu.SemaphoreType.DMA for _ in range(NBUF)],
      ],
  )
  def emb(w_hbm, ids_hbm, out_hbm, idx_v, rows, gsem, osem):
    wid = lax.axis_index("s") * NC + lax.axis_index("c")
    seq0 = wid * SPW
    pltpu.sync_copy(ids_hbm.at[pl.ds(wid * BPW, BPW)], idx_v)

    def gathers(g, bi):
      off = g * T
      return [
          pltpu.make_async_copy(
              w_hbm.at[idx_v.at[pl.ds(off, 128)]],
              rows[bi].at[pl.ds(0, 128)],
              gsem[bi],
          ),
          pltpu.make_async_copy(
              w_hbm.at[idx_v.at[pl.ds(off + 128, 72)]],
              rows[bi].at[pl.ds(128, 72)],
              gsem[bi],
          ),
      ]

    def out_copy(g, bi):
      return pltpu.make_async_copy(
          rows[bi], out_hbm.at[seq0 + g], osem[bi])

    def do_group(g, bi, first=False, start_next=True):
      for d in gathers(g, bi):
        d.wait()
      out_copy(g, bi).start()
      nbi = (bi + 3) % NBUF
      if not first:
        out_copy(g - 1, nbi).wait()
      if start_next:
        for d in gathers(g + 3, nbi):
          d.start()

    for g in range(3):
      for d in gathers(g, g):
        d.start()

    do_group(0, 0, first=True)
    for k in range(1, NBUF):
      do_group(k, k)

    def body(gg, carry):
      g0 = gg * NBUF
      for k in range(NBUF):
        do_group(g0 + k, k)
      return carry

    lax.fori_loop(1, SPW // NBUF - 1, body, 0)

    g0 = SPW - NBUF
    for k in range(NBUF):
      do_group(g0 + k, k, start_next=(g0 + k + 3 < SPW))
    out_copy(SPW - 1, (SPW - 1) % NBUF).wait()

  return emb


def kernel(ids, W):
  ids_flat = ids.reshape(-1).astype(jnp.int32)
  w_padded = _build_widen()(W.T)
  return _build_gather()(w_padded, ids_flat)[:, :, :D]


# speedup vs baseline: 1.6029x; 1.0121x over previous
import functools

import jax
import jax.numpy as jnp
from jax import lax
from jax.experimental import pallas as pl
from jax.experimental.pallas import tpu as pltpu
from jax.experimental.pallas import tpu_sc as plsc

V = 1_000_000
D = 64
DP = 128
NSEQ = 4096
T = 200
B = NSEQ * T
NC, NS = 2, 16
NW = NC * NS
SPW = NSEQ // NW
BPW = SPW * T
NBUF = 4
TBLK = 32768


@functools.cache
def _build_widen():
  def body(wt_ref, out_ref):
    out_ref[:, : D] = jnp.transpose(wt_ref[...], (1, 0))

  return pl.pallas_call(
      body,
      grid=(pl.cdiv(V, TBLK),),
      in_specs=[pl.BlockSpec((D, TBLK), lambda i: (0, i))],
      out_specs=pl.BlockSpec((TBLK, DP), lambda i: (i, 0)),
      out_shape=jax.ShapeDtypeStruct((V, DP), jnp.float32),
  )


@functools.cache
def _build_gather():
  mesh = plsc.VectorSubcoreMesh(core_axis_name="c", subcore_axis_name="s")

  @functools.partial(
      pl.kernel,
      mesh=mesh,
      out_type=jax.ShapeDtypeStruct((NSEQ, T, DP), jnp.float32),
      compiler_params=pltpu.CompilerParams(
          use_tc_tiling_on_sc=True, skip_device_barrier=True),
      scratch_types=[
          pltpu.VMEM((BPW,), jnp.int32),
          [pltpu.VMEM((T, DP), jnp.float32) for _ in range(NBUF)],
          [pltpu.SemaphoreType.DMA for _ in range(NBUF)],
          [pltpu.SemaphoreType.DMA for _ in range(NBUF)],
      ],
  )
  def emb(w_hbm, ids_hbm, out_hbm, idx_v, rows, gsem, osem):
    wid = lax.axis_index("s") * NC + lax.axis_index("c")
    seq0 = wid * SPW
    pltpu.sync_copy(ids_hbm.at[pl.ds(wid * BPW, BPW)], idx_v)

    def gathers(g, bi):
      off = g * T
      return [
          pltpu.make_async_copy(
              w_hbm.at[idx_v.at[pl.ds(off, 128)]],
              rows[bi].at[pl.ds(0, 128)],
              gsem[bi],
          ),
          pltpu.make_async_copy(
              w_hbm.at[idx_v.at[pl.ds(off + 128, 72)]],
              rows[bi].at[pl.ds(128, 72)],
              gsem[bi],
          ),
      ]

    def out_copy(g, bi):
      return pltpu.make_async_copy(
          rows[bi], out_hbm.at[seq0 + g], osem[bi])

    def do_group(g, bi, first=False, start_next=True):
      for d in gathers(g, bi):
        d.wait()
      out_copy(g, bi).start()
      nbi = (bi + 3) % NBUF
      if not first:
        out_copy(g - 1, nbi).wait()
      if start_next:
        for d in gathers(g + 3, nbi):
          d.start()

    for g in range(3):
      for d in gathers(g, g):
        d.start()

    do_group(0, 0, first=True)
    for k in range(1, NBUF):
      do_group(k, k)

    def body(gg, carry):
      g0 = gg * NBUF
      for k in range(NBUF):
        do_group(g0 + k, k)
      return carry

    lax.fori_loop(1, SPW // NBUF - 1, body, 0)

    g0 = SPW - NBUF
    for k in range(NBUF):
      do_group(g0 + k, k, start_next=(g0 + k + 3 < SPW))
    out_copy(SPW - 1, (SPW - 1) % NBUF).wait()

  return emb


def kernel(ids, W):
  ids_flat = ids.reshape(-1).astype(jnp.int32)
  w_padded = _build_widen()(W.T)
  return _build_gather()(w_padded, ids_flat)[:, :, :D]
